# Initial kernel scaffold; baseline (speedup 1.0000x reference)
import jax
import jax.numpy as jnp
from jax.experimental import pallas as pl


def kernel(x, edge_index, W1l, b1, W1r, W2l, b2, W2r):
    raise NotImplementedError("write your pallas kernel here")


# speedup vs baseline: 7.5701x; 7.5701x over previous
import functools

import jax
import jax.numpy as jnp
from jax import lax
from jax.experimental import pallas as pl
from jax.experimental.pallas import tpu as pltpu
from jax.experimental.pallas import tpu_sc as plsc

N = 10000
D = 128
HD = D // 2
NC = 2
NS = 16
BLK = 128
NROWS = 10240
STRIPE = NROWS // NS
DW = 16


@functools.lru_cache(maxsize=None)
def _make_sc_pass(nblk: int):
  mesh = plsc.VectorSubcoreMesh(core_axis_name="c", subcore_axis_name="s")

  @functools.partial(
      pl.kernel,
      out_type=(
          jax.ShapeDtypeStruct((NC, NROWS, HD), jnp.float32),
          jax.ShapeDtypeStruct((NROWS, DW), jnp.float32),
      ),
      mesh=mesh,
      compiler_params=pltpu.CompilerParams(use_tc_tiling_on_sc=False),
      scratch_types=[
          pltpu.VMEM((nblk, BLK), jnp.int32),
          pltpu.VMEM((nblk, BLK), jnp.int32),
          pltpu.VMEM((2, BLK, HD), jnp.float32),
          pltpu.VMEM((BLK, DW), jnp.float32),
          pltpu.VMEM_SHARED((NROWS, HD), jnp.float32),
          pltpu.VMEM_SHARED((NROWS, DW), jnp.float32),
          pltpu.SemaphoreType.DMA((2,)),
          pltpu.SemaphoreType.DMA((2,)),
          pltpu.SemaphoreType.DMA((2,)),
      ],
  )
  def sc_pass(src4, dst3, table, zrows, zdeg, ones_in, out_acc, out_deg,
              sidx, didx, rows, ones, acc, dacc, gsem, ssem, dsem):
    c = lax.axis_index("c")
    s = lax.axis_index("s")

    pltpu.sync_copy(src4.at[c, s], sidx)
    pltpu.sync_copy(dst3.at[s], didx)
    pltpu.sync_copy(ones_in, ones)
    pltpu.sync_copy(zrows.at[pl.ds(s * STRIPE, STRIPE)],
                    acc.at[pl.ds(s * STRIPE, STRIPE)])
    pltpu.sync_copy(zdeg.at[pl.ds(s * STRIPE, STRIPE)],
                    dacc.at[pl.ds(s * STRIPE, STRIPE)])
    plsc.subcore_barrier()

    pltpu.async_copy(table.at[sidx.at[0]], rows.at[0], gsem.at[0])

    def body(j, carry):
      slot = lax.rem(j, 2)
      nslot = lax.rem(j + 1, 2)
      jm1 = jnp.maximum(j - 1, 0)

      @pl.when(j > 0)
      def _():
        pltpu.make_async_copy(rows.at[nslot], acc.at[didx.at[jm1]],
                              ssem.at[nslot]).wait()

        @pl.when(c == 0)
        def _():
          pltpu.make_async_copy(ones, dacc.at[didx.at[jm1]],
                                dsem.at[nslot]).wait()

      @pl.when(j + 1 < nblk)
      def _():
        pltpu.async_copy(table.at[sidx.at[j + 1]], rows.at[nslot],
                         gsem.at[nslot])

      pltpu.make_async_copy(table.at[sidx.at[j]], rows.at[slot],
                            gsem.at[slot]).wait()
      pltpu.async_copy(rows.at[slot], acc.at[didx.at[j]], ssem.at[slot],
                       add=True)

      @pl.when(c == 0)
      def _():
        pltpu.async_copy(ones, dacc.at[didx.at[j]], dsem.at[slot], add=True)

      return carry

    lax.fori_loop(0, nblk, body, 0)

    lslot = (nblk - 1) % 2
    pltpu.make_async_copy(rows.at[lslot], acc.at[didx.at[nblk - 1]],
                          ssem.at[lslot]).wait()

    @pl.when(c == 0)
    def _():
      pltpu.make_async_copy(ones, dacc.at[didx.at[nblk - 1]],
                            dsem.at[lslot]).wait()

    plsc.subcore_barrier()

    pltpu.sync_copy(acc.at[pl.ds(s * STRIPE, STRIPE)],
                    out_acc.at[c, pl.ds(s * STRIPE, STRIPE)])

    @pl.when(c == 0)
    def _():
      pltpu.sync_copy(dacc.at[pl.ds(s * STRIPE, STRIPE)],
                      out_deg.at[pl.ds(s * STRIPE, STRIPE)])

  return sc_pass


_RB = 1280
_GRID = NROWS // _RB


def _mm2_body(x_ref, wa_ref, wb_ref, ya_ref, yb_ref):
  xb = x_ref[...]
  ya_ref[...] = jnp.dot(xb, wa_ref[...], preferred_element_type=jnp.float32)
  yb_ref[...] = jnp.dot(xb, wb_ref[...], preferred_element_type=jnp.float32)


def _mm2(xp, wa, wb):
  return pl.pallas_call(
      _mm2_body,
      grid=(_GRID,),
      in_specs=[
          pl.BlockSpec((_RB, D), lambda i: (i, 0)),
          pl.BlockSpec((D, D), lambda i: (0, 0)),
          pl.BlockSpec((D, D), lambda i: (0, 0)),
      ],
      out_specs=[
          pl.BlockSpec((_RB, D), lambda i: (i, 0)),
          pl.BlockSpec((_RB, D), lambda i: (i, 0)),
      ],
      out_shape=[
          jax.ShapeDtypeStruct((NROWS, D), jnp.float32),
          jax.ShapeDtypeStruct((NROWS, D), jnp.float32),
      ],
  )(xp, wa, wb)


def _mid_body(acc_ref, deg_ref, z1_ref, scale_ref, b1_ref, wa_ref, wb_ref,
              y2_ref, z2_ref):
  agg = jnp.concatenate([acc_ref[0], acc_ref[1]], axis=1)
  deg = deg_ref[:, 0:1]
  mean = agg / jnp.maximum(deg, 1.0)
  h = jnp.maximum(mean + b1_ref[...] + z1_ref[...], 0.0) * scale_ref[...]
  y2_ref[...] = jnp.dot(h, wa_ref[...], preferred_element_type=jnp.float32)
  z2_ref[...] = jnp.dot(h, wb_ref[...], preferred_element_type=jnp.float32)


def _mid(acc1, deg, z1, scale, b1, wa, wb):
  return pl.pallas_call(
      _mid_body,
      grid=(_GRID,),
      in_specs=[
          pl.BlockSpec((NC, _RB, HD), lambda i: (0, i, 0)),
          pl.BlockSpec((_RB, DW), lambda i: (i, 0)),
          pl.BlockSpec((_RB, D), lambda i: (i, 0)),
          pl.BlockSpec((_RB, D), lambda i: (i, 0)),
          pl.BlockSpec((1, D), lambda i: (0, 0)),
          pl.BlockSpec((D, D), lambda i: (0, 0)),
          pl.BlockSpec((D, D), lambda i: (0, 0)),
      ],
      out_specs=[
          pl.BlockSpec((_RB, D), lambda i: (i, 0)),
          pl.BlockSpec((_RB, D), lambda i: (i, 0)),
      ],
      out_shape=[
          jax.ShapeDtypeStruct((NROWS, D), jnp.float32),
          jax.ShapeDtypeStruct((NROWS, D), jnp.float32),
      ],
  )(acc1, deg, z1, scale, b1, wa, wb)


_RBO = 1000


def _post_body(acc_ref, deg_ref, z2_ref, b2_ref, out_ref):
  agg = jnp.concatenate([acc_ref[0], acc_ref[1]], axis=1)
  deg = deg_ref[:, 0:1]
  out_ref[...] = agg / jnp.maximum(deg, 1.0) + b2_ref[...] + z2_ref[...]


def _post(acc2, deg, z2, b2):
  return pl.pallas_call(
      _post_body,
      grid=(N // _RBO,),
      in_specs=[
          pl.BlockSpec((NC, _RBO, HD), lambda i: (0, i, 0)),
          pl.BlockSpec((_RBO, DW), lambda i: (i, 0)),
          pl.BlockSpec((_RBO, D), lambda i: (i, 0)),
          pl.BlockSpec((1, D), lambda i: (0, 0)),
      ],
      out_specs=pl.BlockSpec((_RBO, D), lambda i: (i, 0)),
      out_shape=jax.ShapeDtypeStruct((N, D), jnp.float32),
  )(acc2, deg, z2, b2)


def _split_cols(y):
  return jnp.concatenate([y[:, :HD], y[:, HD:]], axis=0)


def kernel(x, edge_index, W1l, b1, W1r, W2l, b2, W2r):
  E = edge_index.shape[1]
  nblk = -(-E // (NS * BLK))
  epad = NS * nblk * BLK
  src = edge_index[0]
  dst = edge_index[1]
  src3 = jnp.concatenate(
      [src, jnp.zeros((epad - E,), jnp.int32)]).reshape(NS, nblk, BLK)
  src4 = jnp.stack([src3, src3 + NROWS])
  dst3 = jnp.concatenate(
      [dst, jnp.full((epad - E,), N, jnp.int32)]).reshape(NS, nblk, BLK)
  xp = jnp.pad(x, ((0, NROWS - N), (0, 0)))
  keep = (jax.random.uniform(jax.random.key(42), (N, D)) >= 0.1)
  scale = jnp.pad(keep.astype(jnp.float32) / 0.9, ((0, NROWS - N), (0, 0)))
  zrows = jnp.zeros((NROWS, HD), jnp.float32)
  zdeg = jnp.zeros((NROWS, DW), jnp.float32)
  ones_in = jnp.ones((BLK, DW), jnp.float32)

  sc_pass = _make_sc_pass(nblk)
  y1, z1 = _mm2(xp, W1l.T, W1r.T)
  acc1, deg = sc_pass(src4, dst3, _split_cols(y1), zrows, zdeg, ones_in)
  y2, z2 = _mid(acc1, deg, z1, scale, b1.reshape(1, D), W2l.T, W2r.T)
  acc2, _ = sc_pass(src4, dst3, _split_cols(y2), zrows, zdeg, ones_in)
  return _post(acc2, deg, z2, b2.reshape(1, D))


# speedup vs baseline: 8.7924x; 1.1615x over previous
import functools

import jax
import jax.numpy as jnp
from jax import lax
from jax.experimental import pallas as pl
from jax.experimental.pallas import tpu as pltpu
from jax.experimental.pallas import tpu_sc as plsc

N = 10000
D = 128
HD = D // 2
NC = 2
NS = 16
BLK = 128
NBUF = 3
NROWS = 10240
STRIPE = NROWS // NS
DW = 16


@functools.lru_cache(maxsize=None)
def _make_sc_pass(nblk: int, with_deg: bool):
  mesh = plsc.VectorSubcoreMesh(core_axis_name="c", subcore_axis_name="s")
  nhalf = nblk // 2

  out_type = [jax.ShapeDtypeStruct((NC, NROWS, HD), jnp.float32)]
  scratch = [
      pltpu.VMEM((nblk, BLK), jnp.int32),
      pltpu.VMEM((nblk, BLK), jnp.int32),
      pltpu.VMEM((NBUF, BLK, HD), jnp.float32),
      pltpu.SemaphoreType.DMA((NBUF,)),
      pltpu.SemaphoreType.DMA((NBUF,)),
      pltpu.VMEM_SHARED((NROWS, HD), jnp.float32),
  ]
  if with_deg:
    out_type.append(jax.ShapeDtypeStruct((NC, NROWS, DW), jnp.float32))
    scratch += [
        pltpu.VMEM((BLK, DW), jnp.float32),
        pltpu.VMEM_SHARED((NROWS, DW), jnp.float32),
        pltpu.SemaphoreType.DMA,
    ]

  @functools.partial(
      pl.kernel,
      out_type=tuple(out_type),
      mesh=mesh,
      compiler_params=pltpu.CompilerParams(use_tc_tiling_on_sc=False),
      scratch_types=scratch,
  )
  def sc_pass(src4, dst3, table, zrows, zdeg, ones_in, out_acc, *rest):
    if with_deg:
      out_deg, sidx, didx, rows, gsem, ssem, acc, ones, dacc, dsem = rest
    else:
      sidx, didx, rows, gsem, ssem, acc = rest
    c = lax.axis_index("c")
    s = lax.axis_index("s")

    pltpu.sync_copy(src4.at[c, s], sidx)
    pltpu.sync_copy(dst3.at[s], didx)
    pltpu.sync_copy(zrows.at[pl.ds(s * STRIPE, STRIPE)],
                    acc.at[pl.ds(s * STRIPE, STRIPE)])
    if with_deg:
      pltpu.sync_copy(ones_in, ones)
      pltpu.sync_copy(zdeg.at[pl.ds(s * STRIPE, STRIPE)],
                      dacc.at[pl.ds(s * STRIPE, STRIPE)])
    plsc.subcore_barrier()

    pltpu.async_copy(table.at[sidx.at[0]], rows.at[0], gsem.at[0])
    pltpu.async_copy(table.at[sidx.at[1]], rows.at[1], gsem.at[1])

    def body(j, carry):
      bj = lax.rem(j, NBUF)
      bn = lax.rem(j + 2, NBUF)
      jm1 = jnp.maximum(j - 1, 0)

      @pl.when(j > 0)
      def _():
        pltpu.make_async_copy(rows.at[bn], acc.at[didx.at[jm1]],
                              ssem.at[bn]).wait()

      @pl.when(j + 2 < nblk)
      def _():
        pltpu.async_copy(table.at[sidx.at[j + 2]], rows.at[bn], gsem.at[bn])

      pltpu.make_async_copy(table.at[sidx.at[j]], rows.at[bj],
                            gsem.at[bj]).wait()
      pltpu.async_copy(rows.at[bj], acc.at[didx.at[j]], ssem.at[bj],
                       add=True)

      if with_deg:
        @pl.when((j < nhalf) == (c == 0))
        def _():
          pltpu.async_copy(ones, dacc.at[didx.at[j]], dsem, add=True)

      return carry

    lax.fori_loop(0, nblk, body, 0)

    lb = (nblk - 1) % NBUF
    pltpu.make_async_copy(rows.at[lb], acc.at[didx.at[nblk - 1]],
                          ssem.at[lb]).wait()

    if with_deg:
      ndeg = lax.select(c == 0, nhalf, nblk - nhalf)

      def drain(i, carry):
        pltpu.make_async_copy(ones, dacc.at[didx.at[0]], dsem).wait()
        return carry

      lax.fori_loop(0, ndeg, drain, 0)

    plsc.subcore_barrier()

    pltpu.sync_copy(acc.at[pl.ds(s * STRIPE, STRIPE)],
                    out_acc.at[c, pl.ds(s * STRIPE, STRIPE)])
    if with_deg:
      pltpu.sync_copy(dacc.at[pl.ds(s * STRIPE, STRIPE)],
                      out_deg.at[c, pl.ds(s * STRIPE, STRIPE)])

  return sc_pass


_RB = 1000
_GRID = N // _RB


def _mm2_body(x_ref, wa_ref, wb_ref, ys_ref, z_ref):
  xb = x_ref[...]
  wa = wa_ref[...]
  ys_ref[0] = jnp.dot(xb, wa[:, :HD], preferred_element_type=jnp.float32)
  ys_ref[1] = jnp.dot(xb, wa[:, HD:], preferred_element_type=jnp.float32)
  z_ref[...] = jnp.dot(xb, wb_ref[...], preferred_element_type=jnp.float32)


def _mm2(x, wa, wb):
  return pl.pallas_call(
      _mm2_body,
      grid=(_GRID,),
      in_specs=[
          pl.BlockSpec((_RB, D), lambda i: (i, 0)),
          pl.BlockSpec((D, D), lambda i: (0, 0)),
          pl.BlockSpec((D, D), lambda i: (0, 0)),
      ],
      out_specs=[
          pl.BlockSpec((NC, _RB, HD), lambda i: (0, i, 0)),
          pl.BlockSpec((_RB, D), lambda i: (i, 0)),
      ],
      out_shape=[
          jax.ShapeDtypeStruct((NC, NROWS, HD), jnp.float32),
          jax.ShapeDtypeStruct((NROWS, D), jnp.float32),
      ],
  )(x, wa, wb)


def _mid_body(acc_ref, deg_ref, z1_ref, scale_ref, b1_ref, wa_ref, wb_ref,
              ys_ref, z2_ref):
  agg = jnp.concatenate([acc_ref[0], acc_ref[1]], axis=1)
  deg = jnp.maximum(deg_ref[0, :, 0:1] + deg_ref[1, :, 0:1], 1.0)
  h = jnp.maximum(agg / deg + b1_ref[...] + z1_ref[...], 0.0) * scale_ref[...]
  wa = wa_ref[...]
  ys_ref[0] = jnp.dot(h, wa[:, :HD], preferred_element_type=jnp.float32)
  ys_ref[1] = jnp.dot(h, wa[:, HD:], preferred_element_type=jnp.float32)
  z2_ref[...] = jnp.dot(h, wb_ref[...], preferred_element_type=jnp.float32)


def _mid(acc1, deg, z1, scale, b1, wa, wb):
  return pl.pallas_call(
      _mid_body,
      grid=(_GRID,),
      in_specs=[
          pl.BlockSpec((NC, _RB, HD), lambda i: (0, i, 0)),
          pl.BlockSpec((NC, _RB, DW), lambda i: (0, i, 0)),
          pl.BlockSpec((_RB, D), lambda i: (i, 0)),
          pl.BlockSpec((_RB, D), lambda i: (i, 0)),
          pl.BlockSpec((1, D), lambda i: (0, 0)),
          pl.BlockSpec((D, D), lambda i: (0, 0)),
          pl.BlockSpec((D, D), lambda i: (0, 0)),
      ],
      out_specs=[
          pl.BlockSpec((NC, _RB, HD), lambda i: (0, i, 0)),
          pl.BlockSpec((_RB, D), lambda i: (i, 0)),
      ],
      out_shape=[
          jax.ShapeDtypeStruct((NC, NROWS, HD), jnp.float32),
          jax.ShapeDtypeStruct((NROWS, D), jnp.float32),
      ],
  )(acc1, deg, z1, scale, b1, wa, wb)


def _post_body(acc_ref, deg_ref, z2_ref, b2_ref, out_ref):
  agg = jnp.concatenate([acc_ref[0], acc_ref[1]], axis=1)
  deg = jnp.maximum(deg_ref[0, :, 0:1] + deg_ref[1, :, 0:1], 1.0)
  out_ref[...] = agg / deg + b2_ref[...] + z2_ref[...]


def _post(acc2, deg, z2, b2):
  return pl.pallas_call(
      _post_body,
      grid=(_GRID,),
      in_specs=[
          pl.BlockSpec((NC, _RB, HD), lambda i: (0, i, 0)),
          pl.BlockSpec((NC, _RB, DW), lambda i: (0, i, 0)),
          pl.BlockSpec((_RB, D), lambda i: (i, 0)),
          pl.BlockSpec((1, D), lambda i: (0, 0)),
      ],
      out_specs=pl.BlockSpec((_RB, D), lambda i: (i, 0)),
      out_shape=jax.ShapeDtypeStruct((N, D), jnp.float32),
  )(acc2, deg, z2, b2)


def kernel(x, edge_index, W1l, b1, W1r, W2l, b2, W2r):
  E = edge_index.shape[1]
  nblk = -(-E // (NS * BLK))
  epad = NS * nblk * BLK
  src = edge_index[0]
  dst = edge_index[1]
  src3 = jnp.concatenate(
      [src, jnp.zeros((epad - E,), jnp.int32)]).reshape(NS, nblk, BLK)
  src4 = jnp.stack([src3, src3 + NROWS])
  dst3 = jnp.concatenate(
      [dst, jnp.full((epad - E,), N, jnp.int32)]).reshape(NS, nblk, BLK)
  keep = (jax.random.uniform(jax.random.key(42), (N, D)) >= 0.1)
  scale = keep.astype(jnp.float32) / 0.9
  zrows = jnp.zeros((NROWS, HD), jnp.float32)
  zdeg = jnp.zeros((NROWS, DW), jnp.float32)
  ones_in = jnp.ones((BLK, DW), jnp.float32)

  y1s, z1 = _mm2(x, W1l.T, W1r.T)
  acc1, deg = _make_sc_pass(nblk, True)(
      src4, dst3, y1s.reshape(NC * NROWS, HD), zrows, zdeg, ones_in)
  y2s, z2 = _mid(acc1, deg, z1, scale, b1.reshape(1, D), W2l.T, W2r.T)
  (acc2,) = _make_sc_pass(nblk, False)(
      src4, dst3, y2s.reshape(NC * NROWS, HD), zrows, zdeg, ones_in)
  return _post(acc2, deg, z2, b2.reshape(1, D))


# speedup vs baseline: 8.8177x; 1.0029x over previous
import functools

import jax
import jax.numpy as jnp
from jax import lax
from jax.experimental import pallas as pl
from jax.experimental.pallas import tpu as pltpu
from jax.experimental.pallas import tpu_sc as plsc

N = 10000
D = 128
HD = D // 2
NC = 2
NS = 16
BLK = 128
NBUF = 4
NROWS = 10240
STRIPE = NROWS // NS
DW = 16


@functools.lru_cache(maxsize=None)
def _make_sc_pass(nblk: int, with_deg: bool):
  mesh = plsc.VectorSubcoreMesh(core_axis_name="c", subcore_axis_name="s")
  nhalf = nblk // 2

  out_type = [jax.ShapeDtypeStruct((NC, NROWS, HD), jnp.float32)]
  scratch = [
      pltpu.VMEM((nblk, BLK), jnp.int32),
      pltpu.VMEM((nblk, BLK), jnp.int32),
      pltpu.VMEM((NBUF, BLK, HD), jnp.float32),
      pltpu.SemaphoreType.DMA((NBUF,)),
      pltpu.SemaphoreType.DMA((NBUF,)),
      pltpu.VMEM_SHARED((NROWS, HD), jnp.float32),
  ]
  if with_deg:
    out_type.append(jax.ShapeDtypeStruct((NC, NROWS, DW), jnp.float32))
    scratch += [
        pltpu.VMEM((BLK, DW), jnp.float32),
        pltpu.VMEM_SHARED((NROWS, DW), jnp.float32),
        pltpu.SemaphoreType.DMA,
    ]

  @functools.partial(
      pl.kernel,
      out_type=tuple(out_type),
      mesh=mesh,
      compiler_params=pltpu.CompilerParams(use_tc_tiling_on_sc=False),
      scratch_types=scratch,
  )
  def sc_pass(src4, dst3, table, zrows, zdeg, ones_in, out_acc, *rest):
    if with_deg:
      out_deg, sidx, didx, rows, gsem, ssem, acc, ones, dacc, dsem = rest
    else:
      sidx, didx, rows, gsem, ssem, acc = rest
    c = lax.axis_index("c")
    s = lax.axis_index("s")

    pltpu.sync_copy(src4.at[c, s], sidx)
    pltpu.sync_copy(dst3.at[s], didx)
    pltpu.sync_copy(zrows.at[pl.ds(s * STRIPE, STRIPE)],
                    acc.at[pl.ds(s * STRIPE, STRIPE)])
    if with_deg:
      pltpu.sync_copy(ones_in, ones)
      pltpu.sync_copy(zdeg.at[pl.ds(s * STRIPE, STRIPE)],
                      dacc.at[pl.ds(s * STRIPE, STRIPE)])
    plsc.subcore_barrier()

    for b in range(NBUF - 1):
      pltpu.async_copy(table.at[sidx.at[b]], rows.at[b], gsem.at[b])

    def body(j, carry):
      bj = lax.rem(j, NBUF)
      bn = lax.rem(j + NBUF - 1, NBUF)
      jm1 = jnp.maximum(j - 1, 0)

      @pl.when(j > 0)
      def _():
        pltpu.make_async_copy(rows.at[bn], acc.at[didx.at[jm1]],
                              ssem.at[bn]).wait()

      @pl.when(j + NBUF - 1 < nblk)
      def _():
        pltpu.async_copy(table.at[sidx.at[j + NBUF - 1]], rows.at[bn],
                         gsem.at[bn])

      pltpu.make_async_copy(table.at[sidx.at[j]], rows.at[bj],
                            gsem.at[bj]).wait()
      pltpu.async_copy(rows.at[bj], acc.at[didx.at[j]], ssem.at[bj],
                       add=True)

      if with_deg:
        @pl.when((j < nhalf) == (c == 0))
        def _():
          pltpu.async_copy(ones, dacc.at[didx.at[j]], dsem, add=True)

      return carry

    lax.fori_loop(0, nblk, body, 0)

    lb = (nblk - 1) % NBUF
    pltpu.make_async_copy(rows.at[lb], acc.at[didx.at[nblk - 1]],
                          ssem.at[lb]).wait()

    if with_deg:
      ndeg = lax.select(c == 0, nhalf, nblk - nhalf)

      def drain(i, carry):
        pltpu.make_async_copy(ones, dacc.at[didx.at[0]], dsem).wait()
        return carry

      lax.fori_loop(0, ndeg, drain, 0)

    plsc.subcore_barrier()

    pltpu.sync_copy(acc.at[pl.ds(s * STRIPE, STRIPE)],
                    out_acc.at[c, pl.ds(s * STRIPE, STRIPE)])
    if with_deg:
      pltpu.sync_copy(dacc.at[pl.ds(s * STRIPE, STRIPE)],
                      out_deg.at[c, pl.ds(s * STRIPE, STRIPE)])

  return sc_pass


_RB = 1000
_GRID = N // _RB


def _mm2_body(x_ref, wa_ref, wb_ref, ys_ref, z_ref):
  xb = x_ref[...]
  wa = wa_ref[...]
  ys_ref[0] = jnp.dot(xb, wa[:, :HD], preferred_element_type=jnp.float32)
  ys_ref[1] = jnp.dot(xb, wa[:, HD:], preferred_element_type=jnp.float32)
  z_ref[...] = jnp.dot(xb, wb_ref[...], preferred_element_type=jnp.float32)


def _mm2(x, wa, wb):
  return pl.pallas_call(
      _mm2_body,
      grid=(_GRID,),
      in_specs=[
          pl.BlockSpec((_RB, D), lambda i: (i, 0)),
          pl.BlockSpec((D, D), lambda i: (0, 0)),
          pl.BlockSpec((D, D), lambda i: (0, 0)),
      ],
      out_specs=[
          pl.BlockSpec((NC, _RB, HD), lambda i: (0, i, 0)),
          pl.BlockSpec((_RB, D), lambda i: (i, 0)),
      ],
      out_shape=[
          jax.ShapeDtypeStruct((NC, NROWS, HD), jnp.float32),
          jax.ShapeDtypeStruct((NROWS, D), jnp.float32),
      ],
  )(x, wa, wb)


def _mid_body(acc_ref, deg_ref, z1_ref, scale_ref, b1_ref, wa_ref, wb_ref,
              ys_ref, z2_ref):
  agg = jnp.concatenate([acc_ref[0], acc_ref[1]], axis=1)
  deg = jnp.maximum(deg_ref[0, :, 0:1] + deg_ref[1, :, 0:1], 1.0)
  h = jnp.maximum(agg / deg + b1_ref[...] + z1_ref[...], 0.0) * scale_ref[...]
  wa = wa_ref[...]
  ys_ref[0] = jnp.dot(h, wa[:, :HD], preferred_element_type=jnp.float32)
  ys_ref[1] = jnp.dot(h, wa[:, HD:], preferred_element_type=jnp.float32)
  z2_ref[...] = jnp.dot(h, wb_ref[...], preferred_element_type=jnp.float32)


def _mid(acc1, deg, z1, scale, b1, wa, wb):
  return pl.pallas_call(
      _mid_body,
      grid=(_GRID,),
      in_specs=[
          pl.BlockSpec((NC, _RB, HD), lambda i: (0, i, 0)),
          pl.BlockSpec((NC, _RB, DW), lambda i: (0, i, 0)),
          pl.BlockSpec((_RB, D), lambda i: (i, 0)),
          pl.BlockSpec((_RB, D), lambda i: (i, 0)),
          pl.BlockSpec((1, D), lambda i: (0, 0)),
          pl.BlockSpec((D, D), lambda i: (0, 0)),
          pl.BlockSpec((D, D), lambda i: (0, 0)),
      ],
      out_specs=[
          pl.BlockSpec((NC, _RB, HD), lambda i: (0, i, 0)),
          pl.BlockSpec((_RB, D), lambda i: (i, 0)),
      ],
      out_shape=[
          jax.ShapeDtypeStruct((NC, NROWS, HD), jnp.float32),
          jax.ShapeDtypeStruct((NROWS, D), jnp.float32),
      ],
  )(acc1, deg, z1, scale, b1, wa, wb)


def _post_body(acc_ref, deg_ref, z2_ref, b2_ref, out_ref):
  agg = jnp.concatenate([acc_ref[0], acc_ref[1]], axis=1)
  deg = jnp.maximum(deg_ref[0, :, 0:1] + deg_ref[1, :, 0:1], 1.0)
  out_ref[...] = agg / deg + b2_ref[...] + z2_ref[...]


def _post(acc2, deg, z2, b2):
  return pl.pallas_call(
      _post_body,
      grid=(_GRID,),
      in_specs=[
          pl.BlockSpec((NC, _RB, HD), lambda i: (0, i, 0)),
          pl.BlockSpec((NC, _RB, DW), lambda i: (0, i, 0)),
          pl.BlockSpec((_RB, D), lambda i: (i, 0)),
          pl.BlockSpec((1, D), lambda i: (0, 0)),
      ],
      out_specs=pl.BlockSpec((_RB, D), lambda i: (i, 0)),
      out_shape=jax.ShapeDtypeStruct((N, D), jnp.float32),
  )(acc2, deg, z2, b2)


def kernel(x, edge_index, W1l, b1, W1r, W2l, b2, W2r):
  E = edge_index.shape[1]
  nblk = -(-E // (NS * BLK))
  epad = NS * nblk * BLK
  src = edge_index[0]
  dst = edge_index[1]
  src3 = jnp.concatenate(
      [src, jnp.zeros((epad - E,), jnp.int32)]).reshape(NS, nblk, BLK)
  src4 = jnp.stack([src3, src3 + NROWS])
  dst3 = jnp.concatenate(
      [dst, jnp.full((epad - E,), N, jnp.int32)]).reshape(NS, nblk, BLK)
  keep = (jax.random.uniform(jax.random.key(42), (N, D)) >= 0.1)
  scale = keep.astype(jnp.float32) / 0.9
  zrows = jnp.zeros((NROWS, HD), jnp.float32)
  zdeg = jnp.zeros((NROWS, DW), jnp.float32)
  ones_in = jnp.ones((BLK, DW), jnp.float32)

  y1s, z1 = _mm2(x, W1l.T, W1r.T)
  acc1, deg = _make_sc_pass(nblk, True)(
      src4, dst3, y1s.reshape(NC * NROWS, HD), zrows, zdeg, ones_in)
  y2s, z2 = _mid(acc1, deg, z1, scale, b1.reshape(1, D), W2l.T, W2r.T)
  (acc2,) = _make_sc_pass(nblk, False)(
      src4, dst3, y2s.reshape(NC * NROWS, HD), zrows, zdeg, ones_in)
  return _post(acc2, deg, z2, b2.reshape(1, D))


# speedup vs baseline: 11.6945x; 1.3263x over previous
import functools

import jax
import jax.numpy as jnp
from jax import lax
from jax.experimental import pallas as pl
from jax.experimental.pallas import tpu as pltpu
from jax.experimental.pallas import tpu_sc as plsc

N = 10000
D = 128
HD = D // 2
NC = 2
NS = 16
BLK = 128
NBUF = 4
NROWS = 10240
STRIPE = NROWS // NS
DW = 16


@functools.lru_cache(maxsize=None)
def _make_sc_pass(ept: int, with_deg: bool):
  mesh = plsc.VectorSubcoreMesh(core_axis_name="c", subcore_axis_name="s")
  nfull = ept // BLK
  tail = ept - nfull * BLK
  nhalf = nfull // 2

  out_type = [jax.ShapeDtypeStruct((NC, NROWS, HD), jnp.float32)]
  scratch = [
      pltpu.VMEM((ept,), jnp.int32),
      pltpu.VMEM((ept,), jnp.int32),
      pltpu.VMEM((NBUF, BLK, HD), jnp.float32),
      pltpu.VMEM((max(tail, 1), HD), jnp.float32),
      pltpu.SemaphoreType.DMA((NBUF,)),
      pltpu.SemaphoreType.DMA((NBUF,)),
      pltpu.SemaphoreType.DMA,
      pltpu.VMEM_SHARED((NROWS, HD), jnp.float32),
  ]
  if with_deg:
    out_type.append(jax.ShapeDtypeStruct((NC, NROWS, DW), jnp.float32))
    scratch += [
        pltpu.VMEM((BLK, DW), jnp.float32),
        pltpu.VMEM_SHARED((NROWS, DW), jnp.float32),
        pltpu.SemaphoreType.DMA,
        pltpu.SemaphoreType.DMA,
    ]

  @functools.partial(
      pl.kernel,
      out_type=tuple(out_type),
      mesh=mesh,
      compiler_params=pltpu.CompilerParams(use_tc_tiling_on_sc=False),
      scratch_types=scratch,
  )
  def sc_pass(edge_r, table, zrows, zdeg, ones_in, out_acc, *rest):
    if with_deg:
      (out_deg, sidx, didx, rows, rowt, gsem, ssem, tsem, acc,
       ones, dacc, dsem, dsemt) = rest
    else:
      sidx, didx, rows, rowt, gsem, ssem, tsem, acc = rest
    c = lax.axis_index("c")
    s = lax.axis_index("s")

    tab_c = table.at[c]
    pltpu.sync_copy(edge_r.at[0, s], sidx)
    pltpu.sync_copy(edge_r.at[1, s], didx)
    pltpu.sync_copy(zrows.at[pl.ds(s * STRIPE, STRIPE)],
                    acc.at[pl.ds(s * STRIPE, STRIPE)])
    if with_deg:
      pltpu.sync_copy(ones_in, ones)
      pltpu.sync_copy(zdeg.at[pl.ds(s * STRIPE, STRIPE)],
                      dacc.at[pl.ds(s * STRIPE, STRIPE)])
    plsc.subcore_barrier()

    if tail:
      tidx_s = sidx.at[pl.ds(nfull * BLK, tail)]
      tidx_d = didx.at[pl.ds(nfull * BLK, tail)]
      pltpu.async_copy(tab_c.at[tidx_s], rowt, tsem)
      pltpu.make_async_copy(tab_c.at[tidx_s], rowt, tsem).wait()
      pltpu.async_copy(rowt, acc.at[tidx_d], tsem, add=True)
      if with_deg:
        @pl.when(c == 1)
        def _():
          pltpu.async_copy(ones.at[pl.ds(0, tail)], dacc.at[tidx_d],
                           dsemt, add=True)

    for b in range(NBUF - 1):
      pltpu.async_copy(tab_c.at[sidx.at[pl.ds(b * BLK, BLK)]],
                       rows.at[b], gsem.at[b])

    def body(j, carry):
      bj = lax.rem(j, NBUF)
      bn = lax.rem(j + NBUF - 1, NBUF)
      jm1 = jnp.maximum(j - 1, 0)

      @pl.when(j > 0)
      def _():
        pltpu.make_async_copy(rows.at[bn],
                              acc.at[didx.at[pl.ds(jm1 * BLK, BLK)]],
                              ssem.at[bn]).wait()

      @pl.when(j + NBUF - 1 < nfull)
      def _():
        pltpu.async_copy(
            tab_c.at[sidx.at[pl.ds((j + NBUF - 1) * BLK, BLK)]],
            rows.at[bn], gsem.at[bn])

      pltpu.make_async_copy(tab_c.at[sidx.at[pl.ds(j * BLK, BLK)]],
                            rows.at[bj], gsem.at[bj]).wait()
      pltpu.async_copy(rows.at[bj], acc.at[didx.at[pl.ds(j * BLK, BLK)]],
                       ssem.at[bj], add=True)

      if with_deg:
        @pl.when((j < nhalf) == (c == 0))
        def _():
          pltpu.async_copy(ones, dacc.at[didx.at[pl.ds(j * BLK, BLK)]],
                           dsem, add=True)

      return carry

    lax.fori_loop(0, nfull, body, 0)

    lb = (nfull - 1) % NBUF
    pltpu.make_async_copy(rows.at[lb],
                          acc.at[didx.at[pl.ds((nfull - 1) * BLK, BLK)]],
                          ssem.at[lb]).wait()
    if tail:
      pltpu.make_async_copy(rowt, acc.at[didx.at[pl.ds(0, tail)]],
                            tsem).wait()

    if with_deg:
      ndeg = lax.select(c == 0, nhalf, nfull - nhalf)

      def drain(i, carry):
        pltpu.make_async_copy(ones, dacc.at[didx.at[pl.ds(0, BLK)]],
                              dsem).wait()
        return carry

      lax.fori_loop(0, ndeg, drain, 0)
      if tail:
        @pl.when(c == 1)
        def _():
          pltpu.make_async_copy(ones.at[pl.ds(0, tail)],
                                dacc.at[didx.at[pl.ds(0, tail)]],
                                dsemt).wait()

    plsc.subcore_barrier()

    pltpu.sync_copy(acc.at[pl.ds(s * STRIPE, STRIPE)],
                    out_acc.at[c, pl.ds(s * STRIPE, STRIPE)])
    if with_deg:
      pltpu.sync_copy(dacc.at[pl.ds(s * STRIPE, STRIPE)],
                      out_deg.at[c, pl.ds(s * STRIPE, STRIPE)])

  return sc_pass


_RB = 1000
_GRID = N // _RB


def _mm2_body(x_ref, wa_ref, wb_ref, ys_ref, z_ref):
  xb = x_ref[...]
  wa = wa_ref[...]
  ys_ref[0] = jnp.dot(xb, wa[:, :HD], preferred_element_type=jnp.float32)
  ys_ref[1] = jnp.dot(xb, wa[:, HD:], preferred_element_type=jnp.float32)
  z_ref[...] = jnp.dot(xb, wb_ref[...], preferred_element_type=jnp.float32)


def _mm2(x, wa, wb):
  return pl.pallas_call(
      _mm2_body,
      grid=(_GRID,),
      in_specs=[
          pl.BlockSpec((_RB, D), lambda i: (i, 0)),
          pl.BlockSpec((D, D), lambda i: (0, 0)),
          pl.BlockSpec((D, D), lambda i: (0, 0)),
      ],
      out_specs=[
          pl.BlockSpec((NC, _RB, HD), lambda i: (0, i, 0)),
          pl.BlockSpec((_RB, D), lambda i: (i, 0)),
      ],
      out_shape=[
          jax.ShapeDtypeStruct((NC, NROWS, HD), jnp.float32),
          jax.ShapeDtypeStruct((NROWS, D), jnp.float32),
      ],
  )(x, wa, wb)


def _mid_body(acc_ref, deg_ref, z1_ref, scale_ref, b1_ref, wa_ref, wb_ref,
              ys_ref, z2_ref):
  agg = jnp.concatenate([acc_ref[0], acc_ref[1]], axis=1)
  deg = jnp.maximum(deg_ref[0, :, 0:1] + deg_ref[1, :, 0:1], 1.0)
  h = jnp.maximum(agg / deg + b1_ref[...] + z1_ref[...], 0.0) * scale_ref[...]
  wa = wa_ref[...]
  ys_ref[0] = jnp.dot(h, wa[:, :HD], preferred_element_type=jnp.float32)
  ys_ref[1] = jnp.dot(h, wa[:, HD:], preferred_element_type=jnp.float32)
  z2_ref[...] = jnp.dot(h, wb_ref[...], preferred_element_type=jnp.float32)


def _mid(acc1, deg, z1, scale, b1, wa, wb):
  return pl.pallas_call(
      _mid_body,
      grid=(_GRID,),
      in_specs=[
          pl.BlockSpec((NC, _RB, HD), lambda i: (0, i, 0)),
          pl.BlockSpec((NC, _RB, DW), lambda i: (0, i, 0)),
          pl.BlockSpec((_RB, D), lambda i: (i, 0)),
          pl.BlockSpec((_RB, D), lambda i: (i, 0)),
          pl.BlockSpec((1, D), lambda i: (0, 0)),
          pl.BlockSpec((D, D), lambda i: (0, 0)),
          pl.BlockSpec((D, D), lambda i: (0, 0)),
      ],
      out_specs=[
          pl.BlockSpec((NC, _RB, HD), lambda i: (0, i, 0)),
          pl.BlockSpec((_RB, D), lambda i: (i, 0)),
      ],
      out_shape=[
          jax.ShapeDtypeStruct((NC, NROWS, HD), jnp.float32),
          jax.ShapeDtypeStruct((NROWS, D), jnp.float32),
      ],
  )(acc1, deg, z1, scale, b1, wa, wb)


def _post_body(acc_ref, deg_ref, z2_ref, b2_ref, out_ref):
  agg = jnp.concatenate([acc_ref[0], acc_ref[1]], axis=1)
  deg = jnp.maximum(deg_ref[0, :, 0:1] + deg_ref[1, :, 0:1], 1.0)
  out_ref[...] = agg / deg + b2_ref[...] + z2_ref[...]


def _post(acc2, deg, z2, b2):
  return pl.pallas_call(
      _post_body,
      grid=(_GRID,),
      in_specs=[
          pl.BlockSpec((NC, _RB, HD), lambda i: (0, i, 0)),
          pl.BlockSpec((NC, _RB, DW), lambda i: (0, i, 0)),
          pl.BlockSpec((_RB, D), lambda i: (i, 0)),
          pl.BlockSpec((1, D), lambda i: (0, 0)),
      ],
      out_specs=pl.BlockSpec((_RB, D), lambda i: (i, 0)),
      out_shape=jax.ShapeDtypeStruct((N, D), jnp.float32),
  )(acc2, deg, z2, b2)


def kernel(x, edge_index, W1l, b1, W1r, W2l, b2, W2r):
  E = edge_index.shape[1]
  assert E % NS == 0
  ept = E // NS
  edge_r = edge_index.reshape(2, NS, ept)
  keep = (jax.random.uniform(jax.random.key(42), (N, D)) >= 0.1)
  scale = keep.astype(jnp.float32) / 0.9
  zrows = jnp.zeros((NROWS, HD), jnp.float32)
  zdeg = jnp.zeros((NROWS, DW), jnp.float32)
  ones_in = jnp.ones((BLK, DW), jnp.float32)

  y1s, z1 = _mm2(x, W1l.T, W1r.T)
  acc1, deg = _make_sc_pass(ept, True)(
      edge_r, y1s, zrows, zdeg, ones_in)
  y2s, z2 = _mid(acc1, deg, z1, scale, b1.reshape(1, D), W2l.T, W2r.T)
  (acc2,) = _make_sc_pass(ept, False)(
      edge_r, y2s, zrows, zdeg, ones_in)
  return _post(acc2, deg, z2, b2.reshape(1, D))
